# Initial kernel scaffold; baseline (speedup 1.0000x reference)
import jax
import jax.numpy as jnp
from jax.experimental import pallas as pl


def kernel(x, w0, b0, w1, b1, g1, be1, w2, b2, g2, be2, w3, b3, g3, be3, w4, b4, g4, be4, w5, b5, g5, be5, w6, b6):
    raise NotImplementedError("write your pallas kernel here")


# speedup vs baseline: 2.2428x; 2.2428x over previous
import functools

import jax
import jax.numpy as jnp
from jax import lax
from jax.experimental import pallas as pl
from jax.experimental.pallas import tpu as pltpu

_SLOPE = 0.2
_EPS = 1e-5


def _leaky(v):
    return jnp.maximum(v, _SLOPE * v)


def _l1_kernel(p_ref, w_ref, b_ref, o_ref):
    acc = jnp.dot(p_ref[...], w_ref[...], preferred_element_type=jnp.float32)
    acc = acc + b_ref[...]
    o_ref[...] = _leaky(acc)


def _l1_call(patches, wt, b):
    m, k = patches.shape
    o = wt.shape[1]
    nt = 8
    tm = m // nt
    return pl.pallas_call(
        _l1_kernel,
        out_shape=jax.ShapeDtypeStruct((m, o), jnp.float32),
        grid=(nt,),
        in_specs=[
            pl.BlockSpec((tm, k), lambda i: (i, 0)),
            pl.BlockSpec((k, o), lambda i: (0, 0)),
            pl.BlockSpec((1, o), lambda i: (0, 0)),
        ],
        out_specs=pl.BlockSpec((tm, o), lambda i: (i, 0)),
        compiler_params=pltpu.CompilerParams(
            dimension_semantics=("parallel",)),
    )(patches, wt, b.reshape(1, o))


def _tap_build(xq_ref, patch_ref, oh, ow, c, kc):
    m = oh * ow
    for t in range(16):
        kh, kw = t // 4, t % 4
        p, dh = kh % 2, kh // 2
        q, dw = kw % 2, kw // 2
        sl = xq_ref[p, q, dh:dh + oh, dw:dw + ow, :].reshape(m, c)
        col = t * c
        patch_ref[col // kc, :, col % kc:col % kc + c] = sl


def _conv_bn_kernel(xq_ref, w_ref, b_ref, g_ref, be_ref, o_ref, patch_ref, *,
                    oh, ow, c, nk, kc, inv_m):
    k = pl.program_id(1)

    @pl.when(k == 0)
    def _build():
        _tap_build(xq_ref, patch_ref, oh, ow, c, kc)

    acc = jnp.dot(patch_ref[k] if nk > 1 else patch_ref[0],
                  w_ref[...], preferred_element_type=jnp.float32)

    @pl.when(k == 0)
    def _init():
        o_ref[...] = acc

    @pl.when(k > 0)
    def _acc():
        o_ref[...] += acc

    @pl.when(k == nk - 1)
    def _finalize():
        xa = o_ref[...] + b_ref[...]
        mean = jnp.sum(xa, axis=0, keepdims=True) * inv_m
        d = xa - mean
        var = jnp.sum(d * d, axis=0, keepdims=True) * inv_m
        y = d * (g_ref[...] * lax.rsqrt(var + _EPS)) + be_ref[...]
        o_ref[...] = _leaky(y)


def _conv_bn_call(xq, wt, b, g, be, oh, ow, nk):
    _, _, h2, w2, c = xq.shape
    ktot, o = wt.shape
    kc = ktot // nk
    o2 = o // 2
    m = oh * ow
    kern = functools.partial(_conv_bn_kernel, oh=oh, ow=ow, c=c, nk=nk,
                             kc=kc, inv_m=1.0 / m)
    return pl.pallas_call(
        kern,
        out_shape=jax.ShapeDtypeStruct((m, o), jnp.float32),
        grid=(2, nk),
        in_specs=[
            pl.BlockSpec((2, 2, h2, w2, c), lambda j, k: (0, 0, 0, 0, 0)),
            pl.BlockSpec((kc, o2), lambda j, k: (k, j)),
            pl.BlockSpec((1, o2), lambda j, k: (0, j)),
            pl.BlockSpec((1, o2), lambda j, k: (0, j)),
            pl.BlockSpec((1, o2), lambda j, k: (0, j)),
        ],
        out_specs=pl.BlockSpec((m, o2), lambda j, k: (0, j)),
        scratch_shapes=[pltpu.VMEM((nk, m, kc), jnp.float32)],
        compiler_params=pltpu.CompilerParams(
            dimension_semantics=("parallel", "arbitrary")),
    )(xq, wt, b.reshape(1, o), g.reshape(1, o), be.reshape(1, o))


def _head_kernel(xq_ref, w_ref, b_ref, g_ref, be_ref, w7_ref, o_ref,
                 patch_ref, acc_ref, *, oh, ow, c, nk, kc, inv_m):
    k = pl.program_id(1)

    @pl.when(k == 0)
    def _build():
        _tap_build(xq_ref, patch_ref, oh, ow, c, kc)

    acc = jnp.dot(patch_ref[k], w_ref[...], preferred_element_type=jnp.float32)

    @pl.when(k == 0)
    def _init():
        acc_ref[...] = acc

    @pl.when(k > 0)
    def _acc():
        acc_ref[...] += acc

    @pl.when(k == nk - 1)
    def _finalize():
        xa = acc_ref[...] + b_ref[...]
        mean = jnp.sum(xa, axis=0, keepdims=True) * inv_m
        d = xa - mean
        var = jnp.sum(d * d, axis=0, keepdims=True) * inv_m
        y = d * (g_ref[...] * lax.rsqrt(var + _EPS)) + be_ref[...]
        y = _leaky(y)
        o_ref[...] = jnp.sum(y * w7_ref[...]).reshape(1, 1, 1)


def _head_call(xq, wt, b, g, be, w7r, nk):
    _, _, h2, w2, c = xq.shape
    ktot, o = wt.shape
    kc = ktot // nk
    o2 = o // 2
    oh = ow = 4
    m = oh * ow
    kern = functools.partial(_head_kernel, oh=oh, ow=ow, c=c, nk=nk,
                             kc=kc, inv_m=1.0 / m)
    return pl.pallas_call(
        kern,
        out_shape=jax.ShapeDtypeStruct((2, 1, 1), jnp.float32),
        grid=(2, nk),
        in_specs=[
            pl.BlockSpec((2, 2, h2, w2, c), lambda j, k: (0, 0, 0, 0, 0)),
            pl.BlockSpec((kc, o2), lambda j, k: (k, j)),
            pl.BlockSpec((1, o2), lambda j, k: (0, j)),
            pl.BlockSpec((1, o2), lambda j, k: (0, j)),
            pl.BlockSpec((1, o2), lambda j, k: (0, j)),
            pl.BlockSpec((m, o2), lambda j, k: (0, j)),
        ],
        out_specs=pl.BlockSpec((1, 1, 1), lambda j, k: (j, 0, 0)),
        scratch_shapes=[pltpu.VMEM((nk, m, kc), jnp.float32),
                        pltpu.VMEM((m, o2), jnp.float32)],
        compiler_params=pltpu.CompilerParams(
            dimension_semantics=("parallel", "arbitrary")),
    )(xq, wt, b.reshape(1, o), g.reshape(1, o), be.reshape(1, o), w7r)


def _parity(a, oh, ow, o):
    t = a.reshape(oh, ow, o)
    t = jnp.pad(t, ((1, 1), (1, 1), (0, 0)))
    h2, w2 = (oh + 2) // 2, (ow + 2) // 2
    return t.reshape(h2, 2, w2, 2, o).transpose(1, 3, 0, 2, 4)


def _wt(w):
    o, c = w.shape[0], w.shape[1]
    return w.transpose(2, 3, 1, 0).reshape(16 * c, o)


def kernel(x, w0, b0, w1, b1, g1, be1, w2, b2, g2, be2, w3, b3, g3, be3,
           w4, b4, g4, be4, w5, b5, g5, be5, w6, b6):
    xh = x[0].transpose(1, 2, 0)
    patches = lax.conv_general_dilated_patches(
        xh[None], (4, 4), (2, 2), [(1, 1), (1, 1)],
        dimension_numbers=("NHWC", "HWIO", "NHWC"))[0]
    patches = patches.reshape(128 * 128, 96)
    w1t = w0.transpose(1, 2, 3, 0).reshape(96, 64)
    a = _l1_call(patches, w1t, b0)

    xq = _parity(a, 128, 128, 64)
    w2t = jnp.pad(_wt(w1), ((0, 0), (0, 128)))
    b2p = jnp.pad(b1, (0, 128))
    g2p = jnp.pad(g1, (0, 128))
    be2p = jnp.pad(be1, (0, 128))
    a = _conv_bn_call(xq, w2t, b2p, g2p, be2p, 64, 64, nk=1)[:, :128]

    xq = _parity(a, 64, 64, 128)
    a = _conv_bn_call(xq, _wt(w2), b2, g2, be2, 32, 32, nk=2)
    xq = _parity(a, 32, 32, 256)
    a = _conv_bn_call(xq, _wt(w3), b3, g3, be3, 16, 16, nk=4)
    xq = _parity(a, 16, 16, 512)
    a = _conv_bn_call(xq, _wt(w4), b4, g4, be4, 8, 8, nk=8)

    xq = _parity(a, 8, 8, 512)
    w7r = w6.reshape(512, 4, 4).transpose(1, 2, 0).reshape(16, 512)
    parts = _head_call(xq, _wt(w5), b5, g5, be5, w7r, nk=8)
    logit = jnp.sum(parts) + b6[0]
    return jax.nn.sigmoid(logit).reshape(1, 1, 1, 1)
